# Initial kernel scaffold; baseline (speedup 1.0000x reference)
import jax
import jax.numpy as jnp
from jax.experimental import pallas as pl


def kernel(x, up_edge_index, up_edge_weight, down_edge_index, down_edge_weight, params):
    raise NotImplementedError("write your pallas kernel here")


# speedup vs baseline: 10.1525x; 10.1525x over previous
import functools

import jax
import jax.numpy as jnp
from jax import lax
from jax.experimental import pallas as pl
from jax.experimental.pallas import tpu as pltpu
from jax.experimental.pallas import tpu_sc as plsc

N = 10000
D = 128
E = 320000

NC = 2
NS = 16
L = 16

K = 128
NB = 160
CH = 16
EPT = NB * K
NP = 10240
ROWS_PT = NP // NS

_ROW_BLK = 2000
_NRB = N // _ROW_BLK


def _build_rows(buf, ec, i):
    def _grp(g, _):
        wv = ec[i, pl.ds(g * L, L)]
        for l in range(L):
            e = g * L + l
            wvec = jnp.full((L,), wv[l], jnp.float32)
            for j in range(D // L):
                buf[e, pl.ds(j * L, L)] = wvec
        return 0

    lax.fori_loop(0, K // L, _grp, 0)


def _deg_body(col_hbm, ew_hbm, deg_hbm, ic, ec, buf_a, buf_b,
              sem_a, sem_b, deg_sh):
    c = lax.axis_index("c")
    s = lax.axis_index("s")

    def _z(i, _):
        for j in range(D // L):
            buf_a[i, pl.ds(j * L, L)] = jnp.zeros((L,), jnp.float32)
        return 0

    lax.fori_loop(0, K, _z, 0)
    for k in range(ROWS_PT // K):
        pltpu.sync_copy(buf_a, deg_sh.at[pl.ds(s * ROWS_PT + k * K, K)])
    plsc.subcore_barrier()

    def _wait(i, buf, sem):
        pltpu.make_async_copy(buf, deg_sh.at[ic.at[i]], sem).wait()

    def _chunk(jc, _):
        pltpu.sync_copy(col_hbm.at[c, s, pl.ds(jc * CH, CH)], ic)
        pltpu.sync_copy(ew_hbm.at[c, s, pl.ds(jc * CH, CH)], ec)

        def _pair(k, _2):
            i0 = 2 * k
            _build_rows(buf_a, ec, i0)
            pltpu.async_copy(buf_a, deg_sh.at[ic.at[i0]], sem_a, add=True)
            _build_rows(buf_b, ec, i0 + 1)
            pltpu.async_copy(buf_b, deg_sh.at[ic.at[i0 + 1]], sem_b, add=True)
            _wait(i0, buf_a, sem_a)
            _wait(i0 + 1, buf_b, sem_b)
            return 0

        lax.fori_loop(0, CH // 2, _pair, 0)
        return 0

    lax.fori_loop(0, NB // CH, _chunk, 0)
    plsc.subcore_barrier()

    pltpu.sync_copy(deg_sh.at[pl.ds(s * ROWS_PT, ROWS_PT)],
                    deg_hbm.at[pl.ds(c * NP + s * ROWS_PT, ROWS_PT)])


_deg_kernel = pl.kernel(
    _deg_body,
    out_type=jax.ShapeDtypeStruct((NC * NP, D), jnp.float32),
    mesh=plsc.VectorSubcoreMesh(core_axis_name="c", subcore_axis_name="s"),
    scratch_types=[
        pltpu.VMEM((CH, K), jnp.int32),
        pltpu.VMEM((CH, K), jnp.float32),
        pltpu.VMEM((K, D), jnp.float32),
        pltpu.VMEM((K, D), jnp.float32),
        pltpu.SemaphoreType.DMA,
        pltpu.SemaphoreType.DMA,
        pltpu.VMEM_SHARED((NP, D), jnp.float32),
    ],
)


def _scale_rows(buf, ec, i):
    def _grp(g, _):
        wv = ec[i, pl.ds(g * L, L)]
        for l in range(L):
            e = g * L + l
            wvec = jnp.full((L,), wv[l], jnp.float32)
            for j in range(D // L):
                sl = pl.ds(j * L, L)
                buf[e, sl] = buf[e, sl] * wvec
        return 0

    lax.fori_loop(0, K // L, _grp, 0)


def _seg_body(hp_hbm, row_hbm, col_hbm, ew_hbm, s_hbm,
              ir, ic, ec, buf_a, buf_b, sem_a, sem_b, out_sh):
    c = lax.axis_index("c")
    s = lax.axis_index("s")

    def _z(i, _):
        for j in range(D // L):
            buf_a[i, pl.ds(j * L, L)] = jnp.zeros((L,), jnp.float32)
        return 0

    lax.fori_loop(0, K, _z, 0)
    for k in range(ROWS_PT // K):
        pltpu.sync_copy(buf_a, out_sh.at[pl.ds(s * ROWS_PT + k * K, K)])
    plsc.subcore_barrier()

    def _wait(i, buf, sem):
        pltpu.make_async_copy(hp_hbm.at[ir.at[i]], buf, sem).wait()

    def _chunk(jc, _):
        pltpu.sync_copy(row_hbm.at[c, s, pl.ds(jc * CH, CH)], ir)
        pltpu.sync_copy(col_hbm.at[c, s, pl.ds(jc * CH, CH)], ic)
        pltpu.sync_copy(ew_hbm.at[c, s, pl.ds(jc * CH, CH)], ec)
        pltpu.async_copy(hp_hbm.at[ir.at[0]], buf_a, sem_a)

        def _pair(k, _2):
            i0 = 2 * k
            pltpu.async_copy(hp_hbm.at[ir.at[i0 + 1]], buf_b, sem_b)
            _wait(i0, buf_a, sem_a)
            _scale_rows(buf_a, ec, i0)
            pltpu.sync_copy(buf_a, out_sh.at[ic.at[i0]], add=True)

            @pl.when(k < CH // 2 - 1)
            def _():
                pltpu.async_copy(hp_hbm.at[ir.at[i0 + 2]], buf_a, sem_a)

            _wait(i0 + 1, buf_b, sem_b)
            _scale_rows(buf_b, ec, i0 + 1)
            pltpu.sync_copy(buf_b, out_sh.at[ic.at[i0 + 1]], add=True)
            return 0

        lax.fori_loop(0, CH // 2, _pair, 0)
        return 0

    lax.fori_loop(0, NB // CH, _chunk, 0)
    plsc.subcore_barrier()

    pltpu.sync_copy(out_sh.at[pl.ds(s * ROWS_PT, ROWS_PT)],
                    s_hbm.at[pl.ds(c * NP + s * ROWS_PT, ROWS_PT)])


_seg_kernel = pl.kernel(
    _seg_body,
    out_type=jax.ShapeDtypeStruct((NC * NP, D), jnp.float32),
    mesh=plsc.VectorSubcoreMesh(core_axis_name="c", subcore_axis_name="s"),
    scratch_types=[
        pltpu.VMEM((CH, K), jnp.int32),
        pltpu.VMEM((CH, K), jnp.int32),
        pltpu.VMEM((CH, K), jnp.float32),
        pltpu.VMEM((K, D), jnp.float32),
        pltpu.VMEM((K, D), jnp.float32),
        pltpu.SemaphoreType.DMA,
        pltpu.SemaphoreType.DMA,
        pltpu.VMEM_SHARED((NP, D), jnp.float32),
    ],
)


def _prep_body(deg_ref, x_ref, w_ref, dinv_ref, hp_ref):
    deg = deg_ref[:, 0:1] + 1.0
    dinv = jnp.where(deg > 0,
                     lax.rsqrt(jnp.maximum(deg, 1e-12)),
                     jnp.zeros_like(deg))
    h = jnp.dot(x_ref[...], w_ref[0], preferred_element_type=jnp.float32)
    dinv_ref[...] = dinv
    hp_ref[...] = dinv * h


def _prep_call(deg_cat, x, w_cat):
    return pl.pallas_call(
        _prep_body,
        grid=(NC, _NRB),
        in_specs=[
            pl.BlockSpec((_ROW_BLK, L), lambda d, i: (d * _NRB + i, 0)),
            pl.BlockSpec((_ROW_BLK, D), lambda d, i: (i, 0)),
            pl.BlockSpec((1, D, D), lambda d, i: (d, 0, 0)),
        ],
        out_specs=[
            pl.BlockSpec((_ROW_BLK, 1), lambda d, i: (d * _NRB + i, 0)),
            pl.BlockSpec((_ROW_BLK, D), lambda d, i: (d * _NRB + i, 0)),
        ],
        out_shape=[
            jax.ShapeDtypeStruct((NC * N, 1), jnp.float32),
            jax.ShapeDtypeStruct((NC * N, D), jnp.float32),
        ],
    )(deg_cat, x, w_cat)


def _combine(x, su, sd, hpu, hpd, dinvu, dinvd, wg, bg, wu, bu, wd, bd,
             cbu, cbd, lng, lnb):
    hu = dinvu * (su + hpu) + cbu
    hd = dinvd * (sd + hpd) + cbd
    gate = jax.nn.sigmoid(jnp.dot(x, wg, preferred_element_type=jnp.float32) + bg)
    m = gate * (jnp.dot(hu, wu, preferred_element_type=jnp.float32) + bu
                + jnp.dot(hd, wd, preferred_element_type=jnp.float32) + bd)
    r = x + m
    mu = jnp.mean(r, axis=-1, keepdims=True)
    var = jnp.mean((r - mu) ** 2, axis=-1, keepdims=True)
    return (r - mu) * lax.rsqrt(var + 1e-5) * lng + lnb


def _fuse1_body(x_ref, su_ref, sd_ref, hpu_ref, hpd_ref, du_ref, dd_ref,
                wg_ref, bg_ref, wu_ref, bu_ref, wd_ref, bd_ref,
                cbu_ref, cbd_ref, lng_ref, lnb_ref, wc2_ref,
                x1_ref, hp2_ref_u, hp2_ref_d):
    x1 = _combine(x_ref[...], su_ref[...], sd_ref[...], hpu_ref[...],
                  hpd_ref[...], du_ref[...], dd_ref[...],
                  wg_ref[...], bg_ref[...], wu_ref[...], bu_ref[...],
                  wd_ref[...], bd_ref[...], cbu_ref[...], cbd_ref[...],
                  lng_ref[...], lnb_ref[...])
    x1_ref[...] = x1
    hp2_ref_u[...] = du_ref[...] * jnp.dot(
        x1, wc2_ref[0], preferred_element_type=jnp.float32)
    hp2_ref_d[...] = dd_ref[...] * jnp.dot(
        x1, wc2_ref[1], preferred_element_type=jnp.float32)


def _fuse2_body(x_ref, su_ref, sd_ref, hpu_ref, hpd_ref, du_ref, dd_ref,
                wg_ref, bg_ref, wu_ref, bu_ref, wd_ref, bd_ref,
                cbu_ref, cbd_ref, lng_ref, lnb_ref, out_ref):
    out_ref[...] = _combine(x_ref[...], su_ref[...], sd_ref[...],
                            hpu_ref[...], hpd_ref[...], du_ref[...],
                            dd_ref[...], wg_ref[...], bg_ref[...],
                            wu_ref[...], bu_ref[...], wd_ref[...], bd_ref[...],
                            cbu_ref[...], cbd_ref[...], lng_ref[...],
                            lnb_ref[...])


def _row_spec(up):
    del up
    return pl.BlockSpec((_ROW_BLK, D), lambda i: (i, 0))


def _dinv_spec(up):
    del up
    return pl.BlockSpec((_ROW_BLK, 1), lambda i: (i, 0))


_W_SPEC = pl.BlockSpec((D, D), lambda i: (0, 0))
_B_SPEC = pl.BlockSpec((1, D), lambda i: (0, 0))


def _fuse_specs():
    return [
        pl.BlockSpec((_ROW_BLK, D), lambda i: (i, 0)),
        _row_spec(True), _row_spec(False),
        _row_spec(True), _row_spec(False),
        _dinv_spec(True), _dinv_spec(False),
        _W_SPEC, _B_SPEC,
        _W_SPEC, _B_SPEC,
        _W_SPEC, _B_SPEC,
        _B_SPEC, _B_SPEC,
        _B_SPEC, _B_SPEC,
    ]


def _fuse1_call(args, wc2):
    return pl.pallas_call(
        _fuse1_body,
        grid=(_NRB,),
        in_specs=_fuse_specs() + [pl.BlockSpec((NC, D, D), lambda i: (0, 0, 0))],
        out_specs=[
            pl.BlockSpec((_ROW_BLK, D), lambda i: (i, 0)),
            pl.BlockSpec((_ROW_BLK, D), lambda i: (i, 0)),
            pl.BlockSpec((_ROW_BLK, D), lambda i: (i, 0)),
        ],
        out_shape=[
            jax.ShapeDtypeStruct((N, D), jnp.float32),
            jax.ShapeDtypeStruct((N, D), jnp.float32),
            jax.ShapeDtypeStruct((N, D), jnp.float32),
        ],
    )(*args, wc2)


def _fuse2_call(args):
    return pl.pallas_call(
        _fuse2_body,
        grid=(_NRB,),
        in_specs=_fuse_specs(),
        out_specs=pl.BlockSpec((_ROW_BLK, D), lambda i: (i, 0)),
        out_shape=jax.ShapeDtypeStruct((N, D), jnp.float32),
    )(*args)


def _pad_tile(a, fill):
    pad = NS * EPT - E
    return jnp.concatenate(
        [a, jnp.full((pad,), fill, a.dtype)]).reshape(NS, NB, K)


def _dbg_seg(hp, row_cat, col_cat, ew_cat):
    outs = []
    for c in range(NC):
        row = row_cat[c].reshape(-1)
        col = col_cat[c].reshape(-1)
        ew = ew_cat[c].reshape(-1)
        s = jax.ops.segment_sum(hp[row] * ew[:, None], col, num_segments=NP)
        outs.append(s)
    return jnp.concatenate(outs)


def kernel(x, up_edge_index, up_edge_weight, down_edge_index, down_edge_weight,
           params):
    p = params

    rowu = _pad_tile(up_edge_index[0], 0)
    colu = _pad_tile(up_edge_index[1], 0)
    ewu = _pad_tile(up_edge_weight, 0.0)
    rowd = _pad_tile(down_edge_index[0] + N, N)
    cold = _pad_tile(down_edge_index[1], 0)
    ewd = _pad_tile(down_edge_weight, 0.0)

    row_cat = jnp.stack([rowu, rowd])
    col_cat = jnp.stack([colu, cold])
    ew_cat = jnp.stack([ewu, ewd])

    wc1 = jnp.stack([p['up_conv1_w'], p['down_conv1_w']])
    wc2 = jnp.stack([p['up_conv2_w'], p['down_conv2_w']])

    def b2(name):
        return p[name].reshape(1, D)

    deg_pad = _deg_kernel(col_cat, ew_cat)
    deg_cat = jnp.concatenate([deg_pad[:N, :L], deg_pad[NP:NP + N, :L]])

    dinv_cat, hp1 = _prep_call(deg_cat, x, wc1)
    s1 = _seg_kernel(hp1, row_cat, col_cat, ew_cat)
    args1 = (x, s1[:N], s1[NP:NP + N], hp1[:N], hp1[N:], dinv_cat[:N], dinv_cat[N:],
             p['Wg1_w'], b2('Wg1_b'), p['W_up1_w'], b2('W_up1_b'),
             p['W_down1_w'], b2('W_down1_b'), b2('up_conv1_b'),
             b2('down_conv1_b'), b2('ln1_g'), b2('ln1_b'))
    x1, hp2u, hp2d = _fuse1_call(args1, wc2)

    hp2 = jnp.concatenate([hp2u, hp2d])
    s2 = _seg_kernel(hp2, row_cat, col_cat, ew_cat)
    args2 = (x1, s2[:N], s2[NP:NP + N], hp2u, hp2d, dinv_cat[:N], dinv_cat[N:],
             p['Wg2_w'], b2('Wg2_b'), p['W_up2_w'], b2('W_up2_b'),
             p['W_down2_w'], b2('W_down2_b'), b2('up_conv2_b'),
             b2('down_conv2_b'), b2('ln2_g'), b2('ln2_b'))
    return _fuse2_call(args2)


# speedup vs baseline: 10.4169x; 1.0260x over previous
import functools

import jax
import jax.numpy as jnp
from jax import lax
from jax.experimental import pallas as pl
from jax.experimental.pallas import tpu as pltpu
from jax.experimental.pallas import tpu_sc as plsc

N = 10000
D = 128
E = 320000

NC = 2
NS = 16
L = 16

K = 128
NB = 160
CH = 32
EPT = NB * K
NP = 10240
ROWS_PT = NP // NS

_ROW_BLK = 2000
_NRB = N // _ROW_BLK


def _build_rows(buf, ec, i):
    def _grp(g, _):
        wv = ec[i, pl.ds(g * L, L)]
        for l in range(L):
            e = g * L + l
            wvec = _bcast(wv, l)
            for j in range(D // L):
                buf[e, pl.ds(j * L, L)] = wvec
        return 0

    lax.fori_loop(0, K // L, _grp, 0)


def _deg_body(col_hbm, ew_hbm, deg_hbm, ic, ec, buf_a, buf_b,
              sem_a, sem_b, deg_sh):
    c = lax.axis_index("c")
    s = lax.axis_index("s")

    def _z(i, _):
        for j in range(D // L):
            buf_a[i, pl.ds(j * L, L)] = jnp.zeros((L,), jnp.float32)
        return 0

    lax.fori_loop(0, K, _z, 0)
    for k in range(ROWS_PT // K):
        pltpu.sync_copy(buf_a, deg_sh.at[pl.ds(s * ROWS_PT + k * K, K)])
    plsc.subcore_barrier()

    def _wait(i, buf, sem):
        pltpu.make_async_copy(buf, deg_sh.at[ic.at[i]], sem).wait()

    def _chunk(jc, _):
        pltpu.sync_copy(col_hbm.at[c, s, pl.ds(jc * CH, CH)], ic)
        pltpu.sync_copy(ew_hbm.at[c, s, pl.ds(jc * CH, CH)], ec)

        def _pair(k, _2):
            i0 = 2 * k
            _build_rows(buf_a, ec, i0)
            pltpu.async_copy(buf_a, deg_sh.at[ic.at[i0]], sem_a, add=True)
            _build_rows(buf_b, ec, i0 + 1)
            pltpu.async_copy(buf_b, deg_sh.at[ic.at[i0 + 1]], sem_b, add=True)
            _wait(i0, buf_a, sem_a)
            _wait(i0 + 1, buf_b, sem_b)
            return 0

        lax.fori_loop(0, CH // 2, _pair, 0)
        return 0

    lax.fori_loop(0, NB // CH, _chunk, 0)
    plsc.subcore_barrier()

    pltpu.sync_copy(deg_sh.at[pl.ds(s * ROWS_PT, ROWS_PT)],
                    deg_hbm.at[pl.ds(c * NP + s * ROWS_PT, ROWS_PT)])


_deg_kernel = pl.kernel(
    _deg_body,
    out_type=jax.ShapeDtypeStruct((NC * NP, D), jnp.float32),
    mesh=plsc.VectorSubcoreMesh(core_axis_name="c", subcore_axis_name="s"),
    scratch_types=[
        pltpu.VMEM((CH, K), jnp.int32),
        pltpu.VMEM((CH, K), jnp.float32),
        pltpu.VMEM((K, D), jnp.float32),
        pltpu.VMEM((K, D), jnp.float32),
        pltpu.SemaphoreType.DMA,
        pltpu.SemaphoreType.DMA,
        pltpu.VMEM_SHARED((NP, D), jnp.float32),
    ],
)


def _bcast(wv, l):
    return jnp.full((L,), wv[l], jnp.float32)


def _scale_rows(buf, ec, i):
    def _grp(g, _):
        wv = ec[i, pl.ds(g * L, L)]
        for l in range(L):
            e = g * L + l
            wvec = _bcast(wv, l)
            for j in range(D // L):
                sl = pl.ds(j * L, L)
                buf[e, sl] = buf[e, sl] * wvec
        return 0

    lax.fori_loop(0, K // L, _grp, 0)


def _seg_body(hp_hbm, row_hbm, col_hbm, ew_hbm, s_hbm,
              ir, ic, ec, buf_a, buf_b, sem_a, sem_b, ssem_a, ssem_b,
              out_sh):
    c = lax.axis_index("c")
    s = lax.axis_index("s")

    def _z(i, _):
        for j in range(D // L):
            buf_a[i, pl.ds(j * L, L)] = jnp.zeros((L,), jnp.float32)
        return 0

    lax.fori_loop(0, K, _z, 0)
    for k in range(ROWS_PT // K):
        pltpu.sync_copy(buf_a, out_sh.at[pl.ds(s * ROWS_PT + k * K, K)])
    plsc.subcore_barrier()

    def _wait(i, buf, sem):
        pltpu.make_async_copy(hp_hbm.at[ir.at[i]], buf, sem).wait()

    def _chunk(jc, _):
        pltpu.sync_copy(row_hbm.at[c, s, pl.ds(jc * CH, CH)], ir)
        pltpu.sync_copy(col_hbm.at[c, s, pl.ds(jc * CH, CH)], ic)
        pltpu.sync_copy(ew_hbm.at[c, s, pl.ds(jc * CH, CH)], ec)
        pltpu.async_copy(hp_hbm.at[ir.at[0]], buf_a, sem_a)

        def _pair(k, _2):
            i0 = 2 * k
            pltpu.async_copy(hp_hbm.at[ir.at[i0 + 1]], buf_b, sem_b)
            _wait(i0, buf_a, sem_a)
            _scale_rows(buf_a, ec, i0)
            pltpu.sync_copy(buf_a, out_sh.at[ic.at[i0]], add=True)

            @pl.when(k < CH // 2 - 1)
            def _():
                pltpu.async_copy(hp_hbm.at[ir.at[i0 + 2]], buf_a, sem_a)

            _wait(i0 + 1, buf_b, sem_b)
            _scale_rows(buf_b, ec, i0 + 1)
            pltpu.sync_copy(buf_b, out_sh.at[ic.at[i0 + 1]], add=True)
            return 0

        lax.fori_loop(0, CH // 2, _pair, 0)
        return 0

    lax.fori_loop(0, NB // CH, _chunk, 0)
    plsc.subcore_barrier()

    pltpu.sync_copy(out_sh.at[pl.ds(s * ROWS_PT, ROWS_PT)],
                    s_hbm.at[pl.ds(c * NP + s * ROWS_PT, ROWS_PT)])


_seg_kernel = pl.kernel(
    _seg_body,
    out_type=jax.ShapeDtypeStruct((NC * NP, D), jnp.float32),
    mesh=plsc.VectorSubcoreMesh(core_axis_name="c", subcore_axis_name="s"),
    scratch_types=[
        pltpu.VMEM((CH, K), jnp.int32),
        pltpu.VMEM((CH, K), jnp.int32),
        pltpu.VMEM((CH, K), jnp.float32),
        pltpu.VMEM((K, D), jnp.float32),
        pltpu.VMEM((K, D), jnp.float32),
        pltpu.SemaphoreType.DMA,
        pltpu.SemaphoreType.DMA,
        pltpu.SemaphoreType.DMA,
        pltpu.SemaphoreType.DMA,
        pltpu.VMEM_SHARED((NP, D), jnp.float32),
    ],
)


def _prep_body(deg_ref, x_ref, w_ref, dinv_ref, hp_ref):
    deg = deg_ref[:, 0:1] + 1.0
    dinv = jnp.where(deg > 0,
                     lax.rsqrt(jnp.maximum(deg, 1e-12)),
                     jnp.zeros_like(deg))
    h = jnp.dot(x_ref[...], w_ref[0], preferred_element_type=jnp.float32)
    dinv_ref[...] = dinv
    hp_ref[...] = dinv * h


def _prep_call(deg_cat, x, w_cat):
    return pl.pallas_call(
        _prep_body,
        grid=(NC, _NRB),
        in_specs=[
            pl.BlockSpec((_ROW_BLK, L), lambda d, i: (d * _NRB + i, 0)),
            pl.BlockSpec((_ROW_BLK, D), lambda d, i: (i, 0)),
            pl.BlockSpec((1, D, D), lambda d, i: (d, 0, 0)),
        ],
        out_specs=[
            pl.BlockSpec((_ROW_BLK, 1), lambda d, i: (d * _NRB + i, 0)),
            pl.BlockSpec((_ROW_BLK, D), lambda d, i: (d * _NRB + i, 0)),
        ],
        out_shape=[
            jax.ShapeDtypeStruct((NC * N, 1), jnp.float32),
            jax.ShapeDtypeStruct((NC * N, D), jnp.float32),
        ],
    )(deg_cat, x, w_cat)


def _combine(x, su, sd, hpu, hpd, dinvu, dinvd, wg, bg, wu, bu, wd, bd,
             cbu, cbd, lng, lnb):
    hu = dinvu * (su + hpu) + cbu
    hd = dinvd * (sd + hpd) + cbd
    gate = jax.nn.sigmoid(jnp.dot(x, wg, preferred_element_type=jnp.float32) + bg)
    m = gate * (jnp.dot(hu, wu, preferred_element_type=jnp.float32) + bu
                + jnp.dot(hd, wd, preferred_element_type=jnp.float32) + bd)
    r = x + m
    mu = jnp.mean(r, axis=-1, keepdims=True)
    var = jnp.mean((r - mu) ** 2, axis=-1, keepdims=True)
    return (r - mu) * lax.rsqrt(var + 1e-5) * lng + lnb


def _fuse1_body(x_ref, su_ref, sd_ref, hpu_ref, hpd_ref, du_ref, dd_ref,
                wg_ref, bg_ref, wu_ref, bu_ref, wd_ref, bd_ref,
                cbu_ref, cbd_ref, lng_ref, lnb_ref, wc2_ref,
                x1_ref, hp2_ref_u, hp2_ref_d):
    x1 = _combine(x_ref[...], su_ref[...], sd_ref[...], hpu_ref[...],
                  hpd_ref[...], du_ref[...], dd_ref[...],
                  wg_ref[...], bg_ref[...], wu_ref[...], bu_ref[...],
                  wd_ref[...], bd_ref[...], cbu_ref[...], cbd_ref[...],
                  lng_ref[...], lnb_ref[...])
    x1_ref[...] = x1
    hp2_ref_u[...] = du_ref[...] * jnp.dot(
        x1, wc2_ref[0], preferred_element_type=jnp.float32)
    hp2_ref_d[...] = dd_ref[...] * jnp.dot(
        x1, wc2_ref[1], preferred_element_type=jnp.float32)


def _fuse2_body(x_ref, su_ref, sd_ref, hpu_ref, hpd_ref, du_ref, dd_ref,
                wg_ref, bg_ref, wu_ref, bu_ref, wd_ref, bd_ref,
                cbu_ref, cbd_ref, lng_ref, lnb_ref, out_ref):
    out_ref[...] = _combine(x_ref[...], su_ref[...], sd_ref[...],
                            hpu_ref[...], hpd_ref[...], du_ref[...],
                            dd_ref[...], wg_ref[...], bg_ref[...],
                            wu_ref[...], bu_ref[...], wd_ref[...], bd_ref[...],
                            cbu_ref[...], cbd_ref[...], lng_ref[...],
                            lnb_ref[...])


def _row_spec(up):
    del up
    return pl.BlockSpec((_ROW_BLK, D), lambda i: (i, 0))


def _dinv_spec(up):
    del up
    return pl.BlockSpec((_ROW_BLK, 1), lambda i: (i, 0))


_W_SPEC = pl.BlockSpec((D, D), lambda i: (0, 0))
_B_SPEC = pl.BlockSpec((1, D), lambda i: (0, 0))


def _fuse_specs():
    return [
        pl.BlockSpec((_ROW_BLK, D), lambda i: (i, 0)),
        _row_spec(True), _row_spec(False),
        _row_spec(True), _row_spec(False),
        _dinv_spec(True), _dinv_spec(False),
        _W_SPEC, _B_SPEC,
        _W_SPEC, _B_SPEC,
        _W_SPEC, _B_SPEC,
        _B_SPEC, _B_SPEC,
        _B_SPEC, _B_SPEC,
    ]


def _fuse1_call(args, wc2):
    return pl.pallas_call(
        _fuse1_body,
        grid=(_NRB,),
        in_specs=_fuse_specs() + [pl.BlockSpec((NC, D, D), lambda i: (0, 0, 0))],
        out_specs=[
            pl.BlockSpec((_ROW_BLK, D), lambda i: (i, 0)),
            pl.BlockSpec((_ROW_BLK, D), lambda i: (i, 0)),
            pl.BlockSpec((_ROW_BLK, D), lambda i: (i, 0)),
        ],
        out_shape=[
            jax.ShapeDtypeStruct((N, D), jnp.float32),
            jax.ShapeDtypeStruct((N, D), jnp.float32),
            jax.ShapeDtypeStruct((N, D), jnp.float32),
        ],
    )(*args, wc2)


def _fuse2_call(args):
    return pl.pallas_call(
        _fuse2_body,
        grid=(_NRB,),
        in_specs=_fuse_specs(),
        out_specs=pl.BlockSpec((_ROW_BLK, D), lambda i: (i, 0)),
        out_shape=jax.ShapeDtypeStruct((N, D), jnp.float32),
    )(*args)


def _pad_tile(a, fill):
    pad = NS * EPT - E
    return jnp.concatenate(
        [a, jnp.full((pad,), fill, a.dtype)]).reshape(NS, NB, K)


def _dbg_seg(hp, row_cat, col_cat, ew_cat):
    outs = []
    for c in range(NC):
        row = row_cat[c].reshape(-1)
        col = col_cat[c].reshape(-1)
        ew = ew_cat[c].reshape(-1)
        s = jax.ops.segment_sum(hp[row] * ew[:, None], col, num_segments=NP)
        outs.append(s)
    return jnp.concatenate(outs)


def kernel(x, up_edge_index, up_edge_weight, down_edge_index, down_edge_weight,
           params):
    p = params

    rowu = _pad_tile(up_edge_index[0], 0)
    colu = _pad_tile(up_edge_index[1], 0)
    ewu = _pad_tile(up_edge_weight, 0.0)
    rowd = _pad_tile(down_edge_index[0] + N, N)
    cold = _pad_tile(down_edge_index[1], 0)
    ewd = _pad_tile(down_edge_weight, 0.0)

    row_cat = jnp.stack([rowu, rowd])
    col_cat = jnp.stack([colu, cold])
    ew_cat = jnp.stack([ewu, ewd])

    wc1 = jnp.stack([p['up_conv1_w'], p['down_conv1_w']])
    wc2 = jnp.stack([p['up_conv2_w'], p['down_conv2_w']])

    def b2(name):
        return p[name].reshape(1, D)

    deg_pad = _deg_kernel(col_cat, ew_cat)
    deg_cat = jnp.concatenate([deg_pad[:N, :L], deg_pad[NP:NP + N, :L]])

    dinv_cat, hp1 = _prep_call(deg_cat, x, wc1)
    s1 = _seg_kernel(hp1, row_cat, col_cat, ew_cat)
    args1 = (x, s1[:N], s1[NP:NP + N], hp1[:N], hp1[N:], dinv_cat[:N], dinv_cat[N:],
             p['Wg1_w'], b2('Wg1_b'), p['W_up1_w'], b2('W_up1_b'),
             p['W_down1_w'], b2('W_down1_b'), b2('up_conv1_b'),
             b2('down_conv1_b'), b2('ln1_g'), b2('ln1_b'))
    x1, hp2u, hp2d = _fuse1_call(args1, wc2)

    hp2 = jnp.concatenate([hp2u, hp2d])
    s2 = _seg_kernel(hp2, row_cat, col_cat, ew_cat)
    args2 = (x1, s2[:N], s2[NP:NP + N], hp2u, hp2d, dinv_cat[:N], dinv_cat[N:],
             p['Wg2_w'], b2('Wg2_b'), p['W_up2_w'], b2('W_up2_b'),
             p['W_down2_w'], b2('W_down2_b'), b2('up_conv2_b'),
             b2('down_conv2_b'), b2('ln2_g'), b2('ln2_b'))
    return _fuse2_call(args2)
